# baseline (speedup 1.0000x reference)
import jax
import jax.numpy as jnp
from jax import lax
from jax.experimental import pallas as pl
from jax.experimental.pallas import tpu as pltpu
from jax.experimental.pallas import tpu_sc as plsc

N_ROWS = 8192
N_COLS = 2048
NC = 2
NS = 16
NW = NC * NS
ROWS_PER_W = N_ROWS // NW
CHUNK = 8
N_CHUNKS = ROWS_PER_W // CHUNK
N_PAIRS = N_CHUNKS // 2
LANES = 16
COL_VECS = N_COLS // LANES
UNROLL = 4
CHUNK_W = CHUNK * N_COLS


def _permute_body(x_hbm, idx_hbm, out_hbm, idx_v,
                  in_a, in_b, out_a, out_b,
                  sem_ia, sem_ib, sem_oa, sem_ob):
    wid = lax.axis_index("s") * NC + lax.axis_index("c")
    base0 = wid * ROWS_PER_W * N_COLS

    pltpu.sync_copy(idx_hbm, idx_v)

    def in_slice(ci):
        return x_hbm.at[pl.ds(base0 + ci * CHUNK_W, CHUNK_W)]

    def out_slice(ci):
        return out_hbm.at[pl.ds(base0 + ci * CHUNK_W, CHUNK_W)]

    def start_in(ci, buf, sem):
        pltpu.async_copy(in_slice(ci), buf, sem)

    def wait_in(ci, buf, sem):
        pltpu.make_async_copy(in_slice(ci), buf, sem).wait()

    def start_out(ci, buf, sem):
        pltpu.async_copy(buf, out_slice(ci), sem)

    def wait_out(ci, buf, sem):
        pltpu.make_async_copy(buf, out_slice(ci), sem).wait()

    def permute(in_v, out_v):
        for r in range(CHUNK):
            def body(i, _, r=r):
                for u in range(UNROLL):
                    k16 = pl.multiple_of((i * UNROLL + u) * LANES, LANES)
                    src = idx_v[pl.ds(k16, LANES)] + r * N_COLS
                    out_v[pl.ds(r * N_COLS + k16, LANES)] = plsc.load_gather(
                        in_v, [src])
                return 0
            lax.fori_loop(0, COL_VECS // UNROLL, body, 0)

    start_in(0, in_a, sem_ia)
    start_in(1, in_b, sem_ib)

    wait_in(0, in_a, sem_ia)
    permute(in_a, out_a)
    start_out(0, out_a, sem_oa)
    start_in(2, in_a, sem_ia)

    wait_in(1, in_b, sem_ib)
    permute(in_b, out_b)
    start_out(1, out_b, sem_ob)
    start_in(3, in_b, sem_ib)

    def pair_body(g, _):
        ci0 = 2 * g
        ci1 = ci0 + 1

        wait_in(ci0, in_a, sem_ia)
        wait_out(ci0, out_a, sem_oa)
        permute(in_a, out_a)
        start_out(ci0, out_a, sem_oa)
        start_in(jnp.minimum(ci0 + 2, N_CHUNKS - 1), in_a, sem_ia)

        wait_in(ci1, in_b, sem_ib)
        wait_out(ci1, out_b, sem_ob)
        permute(in_b, out_b)
        start_out(ci1, out_b, sem_ob)
        start_in(jnp.minimum(ci1 + 2, N_CHUNKS - 1), in_b, sem_ib)
        return 0

    lax.fori_loop(1, N_PAIRS, pair_body, 0)

    wait_in(N_CHUNKS - 1, in_a, sem_ia)
    wait_in(N_CHUNKS - 1, in_b, sem_ib)
    wait_out(N_CHUNKS - 2, out_a, sem_oa)
    wait_out(N_CHUNKS - 1, out_b, sem_ob)


@jax.jit
def _permute(x_flat, idx):
    kern = pl.kernel(
        _permute_body,
        out_type=jax.ShapeDtypeStruct((N_ROWS * N_COLS,), jnp.float32),
        mesh=plsc.VectorSubcoreMesh(core_axis_name="c", subcore_axis_name="s"),
        scratch_types=[
            pltpu.VMEM((N_COLS,), jnp.int32),
            pltpu.VMEM((CHUNK_W,), jnp.float32),
            pltpu.VMEM((CHUNK_W,), jnp.float32),
            pltpu.VMEM((CHUNK_W,), jnp.float32),
            pltpu.VMEM((CHUNK_W,), jnp.float32),
            pltpu.SemaphoreType.DMA,
            pltpu.SemaphoreType.DMA,
            pltpu.SemaphoreType.DMA,
            pltpu.SemaphoreType.DMA,
        ],
        compiler_params=pltpu.CompilerParams(needs_layout_passes=False),
    )
    return kern(x_flat, idx)


def kernel(x, perm, perm_inv, rev):
    idx = jnp.where(rev, perm_inv, perm).astype(jnp.int32)
    out_flat = _permute(x.reshape(-1), idx)
    return out_flat.reshape(N_ROWS, N_COLS)


# speedup vs baseline: 1.1273x; 1.1273x over previous
import jax
import jax.numpy as jnp
from jax import lax
from jax.experimental import pallas as pl
from jax.experimental.pallas import tpu as pltpu
from jax.experimental.pallas import tpu_sc as plsc

N_ROWS = 8192
N_COLS = 2048
NC = 2
NS = 16
NW = NC * NS
ROWS_PER_W = N_ROWS // NW
CHUNK = 8
N_CHUNKS = ROWS_PER_W // CHUNK
N_PAIRS = N_CHUNKS // 2
LANES = 16
COL_VECS = N_COLS // LANES
UNROLL = 4


def _permute_body(x_hbm, idx_hbm, out_hbm, idx_v,
                  in_a, in_b, out_a, out_b,
                  sem_ia, sem_ib, sem_oa, sem_ob):
    wid = lax.axis_index("s") * NC + lax.axis_index("c")
    row0 = wid * ROWS_PER_W

    pltpu.sync_copy(idx_hbm, idx_v)

    def in_slice(ci):
        return x_hbm.at[pl.ds(row0 + ci * CHUNK, CHUNK), :]

    def out_slice(ci):
        return out_hbm.at[pl.ds(row0 + ci * CHUNK, CHUNK), :]

    def start_in(ci, buf, sem):
        pltpu.async_copy(in_slice(ci), buf, sem)

    def wait_in(ci, buf, sem):
        pltpu.make_async_copy(in_slice(ci), buf, sem).wait()

    def start_out(ci, buf, sem):
        pltpu.async_copy(buf, out_slice(ci), sem)

    def wait_out(ci, buf, sem):
        pltpu.make_async_copy(buf, out_slice(ci), sem).wait()

    def permute(in_v, out_v):
        for r in range(CHUNK):
            row_ids = jnp.zeros((LANES,), jnp.int32) + r
            def body(i, _, r=r, row_ids=row_ids):
                for u in range(UNROLL):
                    k16 = pl.multiple_of((i * UNROLL + u) * LANES, LANES)
                    cols = idx_v[pl.ds(k16, LANES)]
                    out_v[r, pl.ds(k16, LANES)] = plsc.load_gather(
                        in_v, [row_ids, cols])
                return 0
            lax.fori_loop(0, COL_VECS // UNROLL, body, 0)

    start_in(0, in_a, sem_ia)
    start_in(1, in_b, sem_ib)

    wait_in(0, in_a, sem_ia)
    permute(in_a, out_a)
    start_out(0, out_a, sem_oa)
    start_in(2, in_a, sem_ia)

    wait_in(1, in_b, sem_ib)
    permute(in_b, out_b)
    start_out(1, out_b, sem_ob)
    start_in(3, in_b, sem_ib)

    def pair_body(g, _):
        ci0 = 2 * g
        ci1 = ci0 + 1

        wait_in(ci0, in_a, sem_ia)
        wait_out(ci0, out_a, sem_oa)
        permute(in_a, out_a)
        start_out(ci0, out_a, sem_oa)
        start_in(jnp.minimum(ci0 + 2, N_CHUNKS - 1), in_a, sem_ia)

        wait_in(ci1, in_b, sem_ib)
        wait_out(ci1, out_b, sem_ob)
        permute(in_b, out_b)
        start_out(ci1, out_b, sem_ob)
        start_in(jnp.minimum(ci1 + 2, N_CHUNKS - 1), in_b, sem_ib)
        return 0

    lax.fori_loop(1, N_PAIRS, pair_body, 0)

    wait_in(N_CHUNKS - 1, in_a, sem_ia)
    wait_in(N_CHUNKS - 1, in_b, sem_ib)
    wait_out(N_CHUNKS - 2, out_a, sem_oa)
    wait_out(N_CHUNKS - 1, out_b, sem_ob)


@jax.jit
def _permute(x, idx):
    kern = pl.kernel(
        _permute_body,
        out_type=jax.ShapeDtypeStruct((N_ROWS, N_COLS), jnp.float32),
        mesh=plsc.VectorSubcoreMesh(core_axis_name="c", subcore_axis_name="s"),
        scratch_types=[
            pltpu.VMEM((N_COLS,), jnp.int32),
            pltpu.VMEM((CHUNK, N_COLS), jnp.float32),
            pltpu.VMEM((CHUNK, N_COLS), jnp.float32),
            pltpu.VMEM((CHUNK, N_COLS), jnp.float32),
            pltpu.VMEM((CHUNK, N_COLS), jnp.float32),
            pltpu.SemaphoreType.DMA,
            pltpu.SemaphoreType.DMA,
            pltpu.SemaphoreType.DMA,
            pltpu.SemaphoreType.DMA,
        ],
        compiler_params=pltpu.CompilerParams(
            needs_layout_passes=False, use_tc_tiling_on_sc=True),
    )
    return kern(x, idx)


def kernel(x, perm, perm_inv, rev):
    idx = jnp.where(rev, perm_inv, perm).astype(jnp.int32)
    return _permute(x, idx)


# speedup vs baseline: 4.0017x; 3.5497x over previous
import jax
import jax.numpy as jnp
from jax import lax
from jax.experimental import pallas as pl
from jax.experimental.pallas import tpu as pltpu
from jax.experimental.pallas import tpu_sc as plsc

N_ROWS = 8192
N_COLS = 2048
NC = 2
NS = 16
NW = NC * NS
ROWS_PER_W = N_ROWS // NW
CHUNK = 8
N_CHUNKS = ROWS_PER_W // CHUNK
N_PAIRS = N_CHUNKS // 2
LANES = 16
COL_VECS = N_COLS // LANES
UNROLL = 4


def _permute_body(x_hbm, idx_hbm, out_hbm, idx_v,
                  in_a, in_b, out_a, out_b,
                  sem_ia, sem_ib, sem_oa, sem_ob):
    wid = lax.axis_index("s") * NC + lax.axis_index("c")
    row0 = wid * ROWS_PER_W

    pltpu.sync_copy(idx_hbm, idx_v)

    def in_slice(ci):
        return x_hbm.at[pl.ds(row0 + ci * CHUNK, CHUNK), :]

    def out_slice(ci):
        return out_hbm.at[pl.ds(row0 + ci * CHUNK, CHUNK), :]

    def start_in(ci, buf, sem):
        pltpu.async_copy(in_slice(ci), buf, sem)

    def wait_in(ci, buf, sem):
        pltpu.make_async_copy(in_slice(ci), buf, sem).wait()

    def start_out(ci, buf, sem):
        pltpu.async_copy(buf, out_slice(ci), sem)

    def wait_out(ci, buf, sem):
        pltpu.make_async_copy(buf, out_slice(ci), sem).wait()

    def permute(in_v, out_v):
        def body(k, _):
            k16 = pl.multiple_of(k * LANES, LANES)
            cols = idx_v[pl.ds(k16, LANES)]
            vals = []
            for r in range(CHUNK):
                row_ids = jnp.zeros((LANES,), jnp.int32) + r
                vals.append(plsc.load_gather(in_v, [row_ids, cols]))
            for r in range(CHUNK):
                out_v[r, pl.ds(k16, LANES)] = vals[r]
            return 0
        lax.fori_loop(0, COL_VECS, body, 0)

    start_in(0, in_a, sem_ia)
    start_in(1, in_b, sem_ib)

    wait_in(0, in_a, sem_ia)
    permute(in_a, out_a)
    start_out(0, out_a, sem_oa)
    start_in(2, in_a, sem_ia)

    wait_in(1, in_b, sem_ib)
    permute(in_b, out_b)
    start_out(1, out_b, sem_ob)
    start_in(3, in_b, sem_ib)

    def pair_body(g, _):
        ci0 = 2 * g
        ci1 = ci0 + 1

        wait_in(ci0, in_a, sem_ia)
        wait_out(ci0, out_a, sem_oa)
        permute(in_a, out_a)
        start_out(ci0, out_a, sem_oa)
        start_in(jnp.minimum(ci0 + 2, N_CHUNKS - 1), in_a, sem_ia)

        wait_in(ci1, in_b, sem_ib)
        wait_out(ci1, out_b, sem_ob)
        permute(in_b, out_b)
        start_out(ci1, out_b, sem_ob)
        start_in(jnp.minimum(ci1 + 2, N_CHUNKS - 1), in_b, sem_ib)
        return 0

    lax.fori_loop(1, N_PAIRS, pair_body, 0)

    wait_in(N_CHUNKS - 1, in_a, sem_ia)
    wait_in(N_CHUNKS - 1, in_b, sem_ib)
    wait_out(N_CHUNKS - 2, out_a, sem_oa)
    wait_out(N_CHUNKS - 1, out_b, sem_ob)


@jax.jit
def _permute(x, idx):
    kern = pl.kernel(
        _permute_body,
        out_type=jax.ShapeDtypeStruct((N_ROWS, N_COLS), jnp.float32),
        mesh=plsc.VectorSubcoreMesh(core_axis_name="c", subcore_axis_name="s"),
        scratch_types=[
            pltpu.VMEM((N_COLS,), jnp.int32),
            pltpu.VMEM((CHUNK, N_COLS), jnp.float32),
            pltpu.VMEM((CHUNK, N_COLS), jnp.float32),
            pltpu.VMEM((CHUNK, N_COLS), jnp.float32),
            pltpu.VMEM((CHUNK, N_COLS), jnp.float32),
            pltpu.SemaphoreType.DMA,
            pltpu.SemaphoreType.DMA,
            pltpu.SemaphoreType.DMA,
            pltpu.SemaphoreType.DMA,
        ],
        compiler_params=pltpu.CompilerParams(
            needs_layout_passes=False, use_tc_tiling_on_sc=True),
    )
    return kern(x, idx)


def kernel(x, perm, perm_inv, rev):
    idx = jnp.where(rev, perm_inv, perm).astype(jnp.int32)
    return _permute(x, idx)


# speedup vs baseline: 4.0297x; 1.0070x over previous
import jax
import jax.numpy as jnp
from jax import lax
from jax.experimental import pallas as pl
from jax.experimental.pallas import tpu as pltpu
from jax.experimental.pallas import tpu_sc as plsc

N_ROWS = 8192
N_COLS = 2048
NC = 2
NS = 16
NW = NC * NS
ROWS_PER_W = N_ROWS // NW
CHUNK = 8
N_CHUNKS = ROWS_PER_W // CHUNK
N_PAIRS = N_CHUNKS // 2
LANES = 16
COL_VECS = N_COLS // LANES
UNROLL = 4


def _permute_body(x_hbm, idx_hbm, out_hbm, idx_v,
                  in_a, in_b, out_a, out_b,
                  sem_ia, sem_ib, sem_oa, sem_ob):
    wid = lax.axis_index("s") * NC + lax.axis_index("c")
    row0 = wid * ROWS_PER_W

    pltpu.sync_copy(idx_hbm, idx_v)

    def in_slice(ci):
        return x_hbm.at[pl.ds(row0 + ci * CHUNK, CHUNK), :]

    def out_slice(ci):
        return out_hbm.at[pl.ds(row0 + ci * CHUNK, CHUNK), :]

    def start_in(ci, buf, sem):
        pltpu.async_copy(in_slice(ci), buf, sem)

    def wait_in(ci, buf, sem):
        pltpu.make_async_copy(in_slice(ci), buf, sem).wait()

    def start_out(ci, buf, sem):
        pltpu.async_copy(buf, out_slice(ci), sem)

    def wait_out(ci, buf, sem):
        pltpu.make_async_copy(buf, out_slice(ci), sem).wait()

    def permute(in_v, out_v):
        def body(k, _):
            for u in range(2):
                k16 = pl.multiple_of((k * 2 + u) * LANES, LANES)
                cols = idx_v[pl.ds(k16, LANES)]
                vals = []
                for r in range(CHUNK):
                    row_ids = jnp.zeros((LANES,), jnp.int32) + r
                    vals.append(plsc.load_gather(in_v, [row_ids, cols]))
                for r in range(CHUNK):
                    out_v[r, pl.ds(k16, LANES)] = vals[r]
            return 0
        lax.fori_loop(0, COL_VECS // 2, body, 0)

    start_in(0, in_a, sem_ia)
    start_in(1, in_b, sem_ib)

    wait_in(0, in_a, sem_ia)
    permute(in_a, out_a)
    start_out(0, out_a, sem_oa)
    start_in(2, in_a, sem_ia)

    wait_in(1, in_b, sem_ib)
    permute(in_b, out_b)
    start_out(1, out_b, sem_ob)
    start_in(3, in_b, sem_ib)

    def pair_body(g, _):
        ci0 = 2 * g
        ci1 = ci0 + 1

        wait_in(ci0, in_a, sem_ia)
        wait_out(ci0, out_a, sem_oa)
        permute(in_a, out_a)
        start_out(ci0, out_a, sem_oa)
        start_in(jnp.minimum(ci0 + 2, N_CHUNKS - 1), in_a, sem_ia)

        wait_in(ci1, in_b, sem_ib)
        wait_out(ci1, out_b, sem_ob)
        permute(in_b, out_b)
        start_out(ci1, out_b, sem_ob)
        start_in(jnp.minimum(ci1 + 2, N_CHUNKS - 1), in_b, sem_ib)
        return 0

    lax.fori_loop(1, N_PAIRS, pair_body, 0)

    wait_in(N_CHUNKS - 1, in_a, sem_ia)
    wait_in(N_CHUNKS - 1, in_b, sem_ib)
    wait_out(N_CHUNKS - 2, out_a, sem_oa)
    wait_out(N_CHUNKS - 1, out_b, sem_ob)


@jax.jit
def _permute(x, idx):
    kern = pl.kernel(
        _permute_body,
        out_type=jax.ShapeDtypeStruct((N_ROWS, N_COLS), jnp.float32),
        mesh=plsc.VectorSubcoreMesh(core_axis_name="c", subcore_axis_name="s"),
        scratch_types=[
            pltpu.VMEM((N_COLS,), jnp.int32),
            pltpu.VMEM((CHUNK, N_COLS), jnp.float32),
            pltpu.VMEM((CHUNK, N_COLS), jnp.float32),
            pltpu.VMEM((CHUNK, N_COLS), jnp.float32),
            pltpu.VMEM((CHUNK, N_COLS), jnp.float32),
            pltpu.SemaphoreType.DMA,
            pltpu.SemaphoreType.DMA,
            pltpu.SemaphoreType.DMA,
            pltpu.SemaphoreType.DMA,
        ],
        compiler_params=pltpu.CompilerParams(
            needs_layout_passes=False, use_tc_tiling_on_sc=True),
    )
    return kern(x, idx)


def kernel(x, perm, perm_inv, rev):
    idx = jnp.where(rev, perm_inv, perm).astype(jnp.int32)
    return _permute(x, idx)
